# baseline (speedup 1.0000x reference)
import functools
import jax
import jax.numpy as jnp
from jax import lax
from jax.experimental import pallas as pl
from jax.experimental.pallas import tpu as pltpu
from jax.experimental.pallas import tpu_sc as plsc

N_FIELDS = 26
N_PAIRS = N_FIELDS * (N_FIELDS - 1) // 2
D = 128
L = 16
NV = D // L
B = 4096
NC = 2
NS = 16
NW = NC * NS
BPW = B // NW


def _compute_pairs(src, dst):
    row = 0
    for i in range(N_FIELDS - 1):
        a = [src[i, pl.ds(v * L, L)] for v in range(NV)]
        for j in range(i + 1, N_FIELDS):
            for v in range(NV):
                dst[row, pl.ds(v * L, L)] = a[v] * src[j, pl.ds(v * L, L)]
            row += 1


def _sc_body(in_hbm, out_hbm, in_v, out_v, sem_in, sem_o0, sem_o1):
    wid = lax.axis_index("s") * NC + lax.axis_index("c")
    base = wid * BPW
    out_sems = (sem_o0, sem_o1)

    def step(t, c):
        for k in range(2):
            b = base + 2 * t + k
            pltpu.sync_copy(in_hbm.at[b], in_v)

            @pl.when(t > 0)
            def _wait(k=k):
                pltpu.make_async_copy(out_v.at[k], out_hbm.at[b], out_sems[k]).wait()

            _compute_pairs(in_v, out_v.at[k])
            pltpu.make_async_copy(out_v.at[k], out_hbm.at[b], out_sems[k]).start()
        return c

    lax.fori_loop(0, BPW // 2, step, 0)
    last = base + BPW - 1
    pltpu.make_async_copy(out_v.at[0], out_hbm.at[last - 1], sem_o0).wait()
    pltpu.make_async_copy(out_v.at[1], out_hbm.at[last], sem_o1).wait()


def kernel(inputs):
    return pl.kernel(
        _sc_body,
        out_type=jax.ShapeDtypeStruct((B, N_PAIRS, D), jnp.float32),
        mesh=plsc.VectorSubcoreMesh(core_axis_name="c", subcore_axis_name="s"),
        scratch_types=[
            pltpu.VMEM((N_FIELDS, D), jnp.float32),
            pltpu.VMEM((2, N_PAIRS, D), jnp.float32),
            pltpu.SemaphoreType.DMA,
            pltpu.SemaphoreType.DMA,
            pltpu.SemaphoreType.DMA,
        ],
    )(inputs)


# speedup vs baseline: 1.4497x; 1.4497x over previous
import jax
import jax.numpy as jnp
from jax import lax
from jax.experimental import pallas as pl
from jax.experimental.pallas import tpu as pltpu
from jax.experimental.pallas import tpu_sc as plsc

N_FIELDS = 26
N_PAIRS = N_FIELDS * (N_FIELDS - 1) // 2
D = 128
L = 16
NV = D // L
B = 4096
NC = 2
NS = 16
NW = NC * NS
BPW = B // NW


def _compute_pairs(src, dst):
    off = 0
    for i in range(N_FIELDS - 1):
        a = [src[i, pl.ds(v * L, L)] for v in range(NV)]
        lo, hi = i + 1, N_FIELDS
        if hi - lo <= 4:
            for j in range(lo, hi):
                for v in range(NV):
                    dst[off + j - lo, pl.ds(v * L, L)] = a[v] * src[j, pl.ds(v * L, L)]
        else:
            @plsc.parallel_loop(lo, hi, unroll=4)
            def _j(j, a=a, lo=lo, off=off):
                row = off + j - lo
                for v in range(NV):
                    dst[row, pl.ds(v * L, L)] = a[v] * src[j, pl.ds(v * L, L)]
        off += hi - lo


def _sc_body(in_hbm, out_hbm, in_v, out_v, sem_i0, sem_i1, sem_o0, sem_o1):
    wid = lax.axis_index("s") * NC + lax.axis_index("c")
    base = wid * BPW
    in_sems = (sem_i0, sem_i1)
    out_sems = (sem_o0, sem_o1)

    for k in range(2):
        pltpu.make_async_copy(in_hbm.at[base + k], in_v.at[k], in_sems[k]).start()

    def step(t, c):
        for k in range(2):
            b = base + 2 * t + k
            pltpu.make_async_copy(in_hbm.at[b], in_v.at[k], in_sems[k]).wait()

            @pl.when(t > 0)
            def _wait_out(k=k):
                pltpu.make_async_copy(out_v.at[k], out_hbm.at[b], out_sems[k]).wait()

            _compute_pairs(in_v.at[k], out_v.at[k])
            pltpu.make_async_copy(out_v.at[k], out_hbm.at[b], out_sems[k]).start()

            @pl.when(2 * t + k + 2 < BPW)
            def _next_in(k=k, b=b):
                pltpu.make_async_copy(in_hbm.at[b + 2], in_v.at[k], in_sems[k]).start()
        return c

    lax.fori_loop(0, BPW // 2, step, 0)
    last = base + BPW - 1
    pltpu.make_async_copy(out_v.at[0], out_hbm.at[last - 1], sem_o0).wait()
    pltpu.make_async_copy(out_v.at[1], out_hbm.at[last], sem_o1).wait()


def kernel(inputs):
    return pl.kernel(
        _sc_body,
        out_type=jax.ShapeDtypeStruct((B, N_PAIRS, D), jnp.float32),
        mesh=plsc.VectorSubcoreMesh(core_axis_name="c", subcore_axis_name="s"),
        scratch_types=[
            pltpu.VMEM((2, N_FIELDS, D), jnp.float32),
            pltpu.VMEM((2, N_PAIRS, D), jnp.float32),
            pltpu.SemaphoreType.DMA,
            pltpu.SemaphoreType.DMA,
            pltpu.SemaphoreType.DMA,
            pltpu.SemaphoreType.DMA,
        ],
    )(inputs)


# speedup vs baseline: 1.5727x; 1.0848x over previous
import jax
import jax.numpy as jnp
from jax import lax
from jax.experimental import pallas as pl
from jax.experimental.pallas import tpu as pltpu
from jax.experimental.pallas import tpu_sc as plsc

N_FIELDS = 26
N_PAIRS = N_FIELDS * (N_FIELDS - 1) // 2
D = 128
L = 16
NV = D // L
B = 4096
NC = 2
NS = 16
NW = NC * NS
BPW = B // NW
CHUNK_ROWS = (64, 64, 64, 64, 64, 5)
CHUNK_OFF = tuple(sum(CHUNK_ROWS[:c]) for c in range(len(CHUNK_ROWS)))
NCHUNK = len(CHUNK_ROWS)


def _chunk_segments():
    bounds = list(CHUNK_OFF) + [N_PAIRS]
    segs = [[] for _ in range(NCHUNK)]
    row = 0
    for i in range(N_FIELDS - 1):
        j = i + 1
        while j < N_FIELDS:
            c = max(cc for cc in range(NCHUNK) if bounds[cc] <= row)
            take = min(N_FIELDS - j, bounds[c + 1] - row)
            segs[c].append((i, j, j + take, row))
            j += take
            row += take
    return segs


_SEGS = _chunk_segments()


def _compute_chunk(src, dst, segs):
    for i, j_lo, j_hi, dst_row in segs:
        a = [src[i, pl.ds(v * L, L)] for v in range(NV)]
        if j_hi - j_lo <= 2:
            for j in range(j_lo, j_hi):
                for v in range(NV):
                    dst[dst_row + j - j_lo, pl.ds(v * L, L)] = (
                        a[v] * src[j, pl.ds(v * L, L)])
        else:
            @plsc.parallel_loop(j_lo, j_hi, unroll=2)
            def _j(j, a=a, j_lo=j_lo, dst_row=dst_row):
                r = dst_row + j - j_lo
                for v in range(NV):
                    dst[r, pl.ds(v * L, L)] = a[v] * src[j, pl.ds(v * L, L)]


def _sc_body(in_hbm, out_hbm, in_v, ring, sem_i0, sem_i1, sem_o0, sem_o1):
    wid = lax.axis_index("s") * NC + lax.axis_index("c")
    base = wid * BPW
    in_sems = (sem_i0, sem_i1)
    out_sems = (sem_o0, sem_o1)

    def chunk_copy(k, cidx, b, sem):
        return pltpu.make_async_copy(
            ring.at[k, pl.ds(CHUNK_OFF[cidx], CHUNK_ROWS[cidx])],
            out_hbm.at[b, pl.ds(CHUNK_OFF[cidx], CHUNK_ROWS[cidx])],
            sem,
        )

    def drain_group(k):
        for cidx in range(NCHUNK):
            chunk_copy(k, cidx, base, out_sems[k]).wait()

    for k in range(2):
        pltpu.make_async_copy(in_hbm.at[base + k], in_v.at[k], in_sems[k]).start()

    def step(t, c):
        for k in range(2):
            b = base + 2 * t + k
            pltpu.make_async_copy(in_hbm.at[b], in_v.at[k], in_sems[k]).wait()

            @pl.when(2 * t + k + 2 < BPW)
            def _next_in(k=k, b=b):
                pltpu.make_async_copy(in_hbm.at[b + 2], in_v.at[k], in_sems[k]).start()

            @pl.when(t > 0)
            def _drain(k=k):
                drain_group(k)

            for cidx in range(NCHUNK):
                _compute_chunk(in_v.at[k], ring.at[k], _SEGS[cidx])
                chunk_copy(k, cidx, b, out_sems[k]).start()
        return c

    lax.fori_loop(0, BPW // 2, step, 0)
    for k in range(2):
        drain_group(k)


def kernel(inputs):
    return pl.kernel(
        _sc_body,
        out_type=jax.ShapeDtypeStruct((B, N_PAIRS, D), jnp.float32),
        mesh=plsc.VectorSubcoreMesh(core_axis_name="c", subcore_axis_name="s"),
        scratch_types=[
            pltpu.VMEM((2, N_FIELDS, D), jnp.float32),
            pltpu.VMEM((2, N_PAIRS, D), jnp.float32),
            pltpu.SemaphoreType.DMA,
            pltpu.SemaphoreType.DMA,
            pltpu.SemaphoreType.DMA,
            pltpu.SemaphoreType.DMA,
        ],
    )(inputs)
